# baseline (speedup 1.0000x reference)
import functools

import jax
import jax.numpy as jnp
from jax.experimental import pallas as pl
from jax.experimental.pallas import tpu as pltpu

_TAU = 1.0
_ORTHO_LAMBDA = 1e-3
_TILE = 1024
_INV_SQRT2 = 0.7071067811865476
_SQRT2 = 1.4142135623730951


def _moe_kernel(h_ref, q_ref, p_ref, gamma_ref, masks_ref, bias_ref,
                wd_ref, wu_ref, alpha_ref,
                y_ref, ortho_ref, ent_ref,
                wdt_ref, wut_ref,
                *, seq_len, n_rows, n_valid, n_experts, bneck):
    i = pl.program_id(0)
    tile = h_ref.shape[0]

    @pl.when(i == 0)
    def _prep():
        c_up = alpha_ref[0] * (0.5 * _SQRT2)
        for ei in range(n_experts):
            wdt_ref[ei] = (jnp.transpose(wd_ref[ei], (1, 0))
                           * _INV_SQRT2).astype(jnp.bfloat16)
            wut_ref[ei] = (jnp.transpose(wu_ref[ei], (1, 0))
                           * c_up).astype(jnp.bfloat16)
        ent_ref[0] = 0.0
        r = q_ref.shape[1]
        qtq = jax.lax.dot_general(q_ref[:], q_ref[:], (((0,), (0,)), ((), ())),
                                  preferred_element_type=jnp.float32)
        ptp = jax.lax.dot_general(p_ref[:], p_ref[:], (((0,), (0,)), ((), ())),
                                  preferred_element_type=jnp.float32)
        rr = jax.lax.broadcasted_iota(jnp.int32, (r, r), 0)
        cc = jax.lax.broadcasted_iota(jnp.int32, (r, r), 1)
        eye = jnp.where(rr == cc, 1.0, 0.0).astype(jnp.float32)
        ortho_ref[0] = _ORTHO_LAMBDA * (jnp.sum((qtq - eye) ** 2) +
                                        jnp.sum((ptp - eye) ** 2))

    h = h_ref[:]
    h16 = h.astype(jnp.bfloat16)

    z = jnp.dot(h, q_ref[:], preferred_element_type=jnp.float32)
    e = z * z
    e = e / (jnp.sum(e, axis=-1, keepdims=True) + 1e-6)
    m = jax.nn.softmax(masks_ref[:], axis=0)
    w_route = m * gamma_ref[:]
    logits = jax.lax.dot_general(
        e, w_route, (((1,), (1,)), ((), ())),
        preferred_element_type=jnp.float32) + bias_ref[:]
    probs = jax.nn.softmax(logits / _TAU, axis=-1)

    out = None
    for ei in range(n_experts):
        u = jnp.dot(h16, wdt_ref[ei],
                    preferred_element_type=jnp.float32)
        g = u * (1.0 + jax.lax.erf(u))
        g = (g * probs[:, ei:ei + 1]).astype(jnp.bfloat16)
        o = jnp.dot(g, wut_ref[ei],
                    preferred_element_type=jnp.float32)
        out = o if out is None else out + o

    rid = i * tile + jax.lax.broadcasted_iota(jnp.int32, (tile, 1), 0)
    is_patch = jnp.logical_and((rid % seq_len) != 0, rid < n_rows)
    y_ref[:] = jnp.where(is_patch, h + out, h)

    p_ent = -probs * jnp.log(jnp.clip(probs, 1e-9, None))
    row_ent = jnp.sum(p_ent, axis=-1, keepdims=True)
    tile_ent = jnp.sum(jnp.where(is_patch, row_ent, 0.0))
    ent_ref[0] = ent_ref[0] + tile_ent

    @pl.when(i == pl.num_programs(0) - 1)
    def _fin():
        ent_ref[0] = ent_ref[0] / n_valid


def kernel(x, Q, P, gamma, masks, bias, down_w, up_w, alpha):
    b, t, d = x.shape
    n_experts, bneck, _ = down_w.shape
    r = Q.shape[1]
    n_rows = b * t
    n_valid = b * (t - 1)

    h_flat = x.reshape(n_rows, d)
    num_tiles = pl.cdiv(n_rows, _TILE)
    pad = num_tiles * _TILE - n_rows
    if pad:
        h_flat = jnp.pad(h_flat, ((0, pad), (0, 0)))

    gamma2 = gamma.reshape(1, r)
    bias2 = bias.reshape(1, n_experts)
    alpha1 = alpha.reshape(1)

    kern = functools.partial(
        _moe_kernel, seq_len=t, n_rows=n_rows, n_valid=n_valid,
        n_experts=n_experts, bneck=bneck)

    y, ortho, ent = pl.pallas_call(
        kern,
        grid=(num_tiles,),
        in_specs=[
            pl.BlockSpec((_TILE, d), lambda i: (i, 0)),
            pl.BlockSpec((d, r), lambda i: (0, 0)),
            pl.BlockSpec((d, r), lambda i: (0, 0)),
            pl.BlockSpec((1, r), lambda i: (0, 0)),
            pl.BlockSpec((n_experts, r), lambda i: (0, 0)),
            pl.BlockSpec((1, n_experts), lambda i: (0, 0)),
            pl.BlockSpec((n_experts, bneck, d), lambda i: (0, 0, 0)),
            pl.BlockSpec((n_experts, d, bneck), lambda i: (0, 0, 0)),
            pl.BlockSpec(memory_space=pltpu.SMEM),
        ],
        out_specs=[
            pl.BlockSpec((_TILE, d), lambda i: (i, 0)),
            pl.BlockSpec(memory_space=pltpu.SMEM),
            pl.BlockSpec(memory_space=pltpu.SMEM),
        ],
        out_shape=[
            jax.ShapeDtypeStruct((num_tiles * _TILE, d), jnp.float32),
            jax.ShapeDtypeStruct((1,), jnp.float32),
            jax.ShapeDtypeStruct((1,), jnp.float32),
        ],
        scratch_shapes=[
            pltpu.VMEM((n_experts, d, bneck), jnp.bfloat16),
            pltpu.VMEM((n_experts, bneck, d), jnp.bfloat16),
        ],
    )(h_flat, Q, P, gamma2, masks, bias2, down_w, up_w, alpha1)

    y = y[:n_rows].reshape(b, t, d)
    return y, ortho[0], ent[0]


# speedup vs baseline: 1.2175x; 1.2175x over previous
import functools

import jax
import jax.numpy as jnp
from jax.experimental import pallas as pl
from jax.experimental.pallas import tpu as pltpu

_TAU = 1.0
_ORTHO_LAMBDA = 1e-3
_TILE = 1024
_INV_SQRT2 = 0.7071067811865476
_SQRT2 = 1.4142135623730951


def _moe_kernel(h_ref, q_ref, p_ref, gamma_ref, masks_ref, bias_ref,
                wd_ref, wu_ref, expand_ref, alpha_ref,
                y_ref, ortho_ref, ent_ref,
                *, seq_len, n_rows, n_valid, n_experts, bneck):
    i = pl.program_id(0)
    tile = h_ref.shape[0]
    h = h_ref[:]
    h16 = h.astype(jnp.bfloat16)

    z = jnp.dot(h, q_ref[:], preferred_element_type=jnp.float32)
    e = z * z
    e = e / (jnp.sum(e, axis=-1, keepdims=True) + 1e-6)
    m = jax.nn.softmax(masks_ref[:], axis=0)
    w_route = m * gamma_ref[:]
    logits = jax.lax.dot_general(
        e, w_route, (((1,), (1,)), ((), ())),
        preferred_element_type=jnp.float32) + bias_ref[:]
    probs = jax.nn.softmax(logits / _TAU, axis=-1)

    u = jnp.dot(h16, wd_ref[:],
                preferred_element_type=jnp.float32)
    g = u * (1.0 + jax.lax.erf(u))
    probs_wide = jnp.dot(probs.astype(jnp.bfloat16), expand_ref[:],
                         preferred_element_type=jnp.float32)
    weighted = (g * probs_wide).astype(jnp.bfloat16)
    out = jnp.dot(weighted, wu_ref[:], preferred_element_type=jnp.float32)

    rid = i * tile + jax.lax.broadcasted_iota(jnp.int32, (tile, 1), 0)
    is_patch = jnp.logical_and((rid % seq_len) != 0, rid < n_rows)
    y_ref[:] = jnp.where(is_patch, h + out, h)

    p_ent = -probs * jnp.log(jnp.clip(probs, 1e-9, None))
    row_ent = jnp.sum(p_ent, axis=-1, keepdims=True)
    tile_ent = jnp.sum(jnp.where(is_patch, row_ent, 0.0))

    @pl.when(i == 0)
    def _init():
        ent_ref[0] = 0.0
        r = q_ref.shape[1]
        qtq = jax.lax.dot_general(q_ref[:], q_ref[:], (((0,), (0,)), ((), ())),
                                  preferred_element_type=jnp.float32)
        ptp = jax.lax.dot_general(p_ref[:], p_ref[:], (((0,), (0,)), ((), ())),
                                  preferred_element_type=jnp.float32)
        rr = jax.lax.broadcasted_iota(jnp.int32, (r, r), 0)
        cc = jax.lax.broadcasted_iota(jnp.int32, (r, r), 1)
        eye = jnp.where(rr == cc, 1.0, 0.0).astype(jnp.float32)
        ortho_ref[0] = _ORTHO_LAMBDA * (jnp.sum((qtq - eye) ** 2) +
                                        jnp.sum((ptp - eye) ** 2))

    ent_ref[0] = ent_ref[0] + tile_ent

    @pl.when(i == pl.num_programs(0) - 1)
    def _fin():
        ent_ref[0] = ent_ref[0] / n_valid


def kernel(x, Q, P, gamma, masks, bias, down_w, up_w, alpha):
    b, t, d = x.shape
    n_experts, bneck, _ = down_w.shape
    r = Q.shape[1]
    en = n_experts * bneck
    n_rows = b * t
    n_valid = b * (t - 1)

    h_flat = x.reshape(n_rows, d)
    num_tiles = pl.cdiv(n_rows, _TILE)
    pad = num_tiles * _TILE - n_rows
    if pad:
        h_flat = jnp.pad(h_flat, ((0, pad), (0, 0)))

    wd = (down_w.transpose(2, 0, 1).reshape(d, en)
          * _INV_SQRT2).astype(jnp.bfloat16)
    wu = (up_w.transpose(0, 2, 1).reshape(en, d)
          * (alpha * (0.5 * _SQRT2))).astype(jnp.bfloat16)
    eidx = jnp.arange(n_experts)
    cidx = jnp.arange(en) // bneck
    expand = (eidx[:, None] == cidx[None, :]).astype(jnp.bfloat16)
    gamma2 = gamma.reshape(1, r)
    bias2 = bias.reshape(1, n_experts)
    alpha1 = alpha.reshape(1)

    kern = functools.partial(
        _moe_kernel, seq_len=t, n_rows=n_rows, n_valid=n_valid,
        n_experts=n_experts, bneck=bneck)

    y, ortho, ent = pl.pallas_call(
        kern,
        grid=(num_tiles,),
        in_specs=[
            pl.BlockSpec((_TILE, d), lambda i: (i, 0)),
            pl.BlockSpec((d, r), lambda i: (0, 0)),
            pl.BlockSpec((d, r), lambda i: (0, 0)),
            pl.BlockSpec((1, r), lambda i: (0, 0)),
            pl.BlockSpec((n_experts, r), lambda i: (0, 0)),
            pl.BlockSpec((1, n_experts), lambda i: (0, 0)),
            pl.BlockSpec((d, en), lambda i: (0, 0)),
            pl.BlockSpec((en, d), lambda i: (0, 0)),
            pl.BlockSpec((n_experts, en), lambda i: (0, 0)),
            pl.BlockSpec(memory_space=pltpu.SMEM),
        ],
        out_specs=[
            pl.BlockSpec((_TILE, d), lambda i: (i, 0)),
            pl.BlockSpec(memory_space=pltpu.SMEM),
            pl.BlockSpec(memory_space=pltpu.SMEM),
        ],
        out_shape=[
            jax.ShapeDtypeStruct((num_tiles * _TILE, d), jnp.float32),
            jax.ShapeDtypeStruct((1,), jnp.float32),
            jax.ShapeDtypeStruct((1,), jnp.float32),
        ],
    )(h_flat, Q, P, gamma2, masks, bias2, wd, wu, expand, alpha1)

    y = y[:n_rows].reshape(b, t, d)
    return y, ortho[0], ent[0]
